# baseline (speedup 1.0000x reference)
import jax
import jax.numpy as jnp
from jax import lax
from jax.experimental import pallas as pl
from jax.experimental.pallas import tpu as pltpu
from jax.experimental.pallas import tpu_sc as plsc

_NC = 2
_NS = 16
_NW = _NC * _NS
_L = 16

_C2 = 12800


def _sigmoid_chunk(vb, base, n):
    def body(i, c):
        b = base + i * (4 * _L)
        for j in range(4):
            s = pl.ds(b + j * _L, _L)
            x = vb[s]
            vb[s] = 1.0 / (1.0 + jnp.exp(-x))
        return c

    lax.fori_loop(0, n // (4 * _L), body, 0)


def _sc_body(idx_hbm, w_hbm, out_hbm, w_sh,
             idx_a, idx_b, vals_a, vals_b, vals_e,
             sem_a, sem_b, sem_c, sem_d, sem_e):
    cid = lax.axis_index("c")
    sid = lax.axis_index("s")
    wid = sid * _NC + cid

    v = w_hbm.shape[0]
    n_pieces = v // _C2
    rem = v - n_pieces * _C2
    n_uniform = n_pieces // _NS
    n_extra = n_pieces - n_uniform * _NS
    stage_bufs = ((vals_a, sem_a, sem_c), (vals_b, sem_b, sem_d))

    def stage_in(k):
        vb, _, si = stage_bufs[k % 2]
        o = (k * _NS + sid) * _C2
        return pltpu.async_copy(w_hbm.at[pl.ds(o, _C2)], vb, si)

    incps = {0: stage_in(0), 1: stage_in(1)}

    o_x = (n_uniform * _NS + sid) * _C2
    o_r = n_pieces * _C2
    if n_extra > 0:
        @pl.when(sid < n_extra)
        def _():
            pltpu.async_copy(w_hbm.at[pl.ds(o_x, _C2)], vals_e, sem_e)
    if rem > 0:
        @pl.when(sid == _NS - 1)
        def _():
            pltpu.async_copy(w_hbm.at[pl.ds(o_r, rem)],
                             vals_e.at[pl.ds(0, rem)], sem_e)

    outcps = []
    for k in range(n_uniform):
        vb, so, _ = stage_bufs[k % 2]
        incps[k].wait()
        o = (k * _NS + sid) * _C2
        outcps.append(pltpu.async_copy(vb, w_sh.at[pl.ds(o, _C2)], so))
        if k + 2 < n_uniform:
            outcps[k].wait()
            incps[k + 2] = stage_in(k + 2)
    per_w = idx_hbm.shape[0] // _NW
    base = wid * per_w
    pre0 = pltpu.async_copy(idx_hbm.at[pl.ds(base, _C2)], idx_a, sem_c)

    for cp in outcps[-2:]:
        cp.wait()

    if n_extra > 0:
        @pl.when(sid < n_extra)
        def _():
            pltpu.make_async_copy(w_hbm.at[pl.ds(o_x, _C2)], vals_e,
                                  sem_e).wait()
            pltpu.async_copy(vals_e, w_sh.at[pl.ds(o_x, _C2)], sem_e).wait()

    if rem > 0:
        @pl.when(sid == _NS - 1)
        def _():
            pltpu.make_async_copy(w_hbm.at[pl.ds(o_r, rem)],
                                  vals_e.at[pl.ds(0, rem)], sem_e).wait()
            pltpu.async_copy(vals_e.at[pl.ds(0, rem)],
                             w_sh.at[pl.ds(o_r, rem)], sem_e).wait()
    plsc.subcore_barrier()

    nch = per_w // _C2
    bufs = ((idx_a, vals_a, sem_a), (idx_b, vals_b, sem_b))

    pre0.wait()
    pending = (pltpu.async_copy(w_sh.at[idx_a], vals_a, sem_a), base, vals_a)
    for g in range(1, nch - 1):
        ib, vb, sm = bufs[g % 2]
        off = base + g * _C2
        pltpu.sync_copy(idx_hbm.at[pl.ds(off, _C2)], ib)
        cp = pltpu.async_copy(w_sh.at[ib], vb, sm)
        pcp, poff, pvb = pending
        pcp.wait()
        _sigmoid_chunk(pvb, 0, _C2)
        pltpu.sync_copy(pvb, out_hbm.at[pl.ds(poff, _C2)])
        pending = (cp, off, vb)

    h = _C2 // 2
    off7 = base + (nch - 1) * _C2
    pltpu.sync_copy(idx_hbm.at[pl.ds(off7, _C2)], idx_b)
    cp7a = pltpu.async_copy(w_sh.at[idx_b.at[pl.ds(0, h)]],
                            vals_b.at[pl.ds(0, h)], sem_b)
    cp7b = pltpu.async_copy(w_sh.at[idx_b.at[pl.ds(h, h)]],
                            vals_e.at[pl.ds(0, h)], sem_e)
    pcp, poff, pvb = pending
    pcp.wait()
    _sigmoid_chunk(pvb, 0, _C2)
    pltpu.sync_copy(pvb, out_hbm.at[pl.ds(poff, _C2)])
    cp7a.wait()
    _sigmoid_chunk(vals_b, 0, h)
    pltpu.sync_copy(vals_b.at[pl.ds(0, h)], out_hbm.at[pl.ds(off7, h)])
    cp7b.wait()
    _sigmoid_chunk(vals_e, 0, h)
    pltpu.sync_copy(vals_e.at[pl.ds(0, h)], out_hbm.at[pl.ds(off7 + h, h)])


def kernel(idx, weight):
    n = idx.shape[0]
    assert n % (_NW * _C2) == 0
    assert weight.shape[0] % 8 == 0
    flat_idx = idx.reshape(-1)

    run = pl.kernel(
        _sc_body,
        out_type=jax.ShapeDtypeStruct((n,), jnp.float32),
        mesh=plsc.VectorSubcoreMesh(core_axis_name="c", subcore_axis_name="s"),
        scratch_types=[
            pltpu.VMEM_SHARED((weight.shape[0],), jnp.float32),
            pltpu.VMEM((_C2,), jnp.int32),
            pltpu.VMEM((_C2,), jnp.int32),
            pltpu.VMEM((_C2,), jnp.float32),
            pltpu.VMEM((_C2,), jnp.float32),
            pltpu.VMEM((_C2,), jnp.float32),
            pltpu.SemaphoreType.DMA,
            pltpu.SemaphoreType.DMA,
            pltpu.SemaphoreType.DMA,
            pltpu.SemaphoreType.DMA,
            pltpu.SemaphoreType.DMA,
        ],
    )
    out = run(flat_idx, weight)
    return out.reshape(idx.shape)


# speedup vs baseline: 1.0031x; 1.0031x over previous
import jax
import jax.numpy as jnp
from jax import lax
from jax.experimental import pallas as pl
from jax.experimental.pallas import tpu as pltpu
from jax.experimental.pallas import tpu_sc as plsc

_NC = 2
_NS = 16
_NW = _NC * _NS
_L = 16

_C2 = 12800


def _sigmoid_chunk(vb, n):
    def body(i, c):
        b = i * (4 * _L)
        for j in range(4):
            s = pl.ds(b + j * _L, _L)
            x = vb[s]
            vb[s] = 1.0 / (1.0 + jnp.exp(-x))
        return c

    lax.fori_loop(0, n // (4 * _L), body, 0)


def _sc_body(idx_hbm, w_hbm, out_hbm, w_sh,
             idx_a, idx_b, vals_a, vals_b, vals_e,
             sem_a, sem_b, sem_c, sem_d, sem_e):
    cid = lax.axis_index("c")
    sid = lax.axis_index("s")
    wid = sid * _NC + cid

    v = w_hbm.shape[0]
    n_pieces = v // _C2
    rem = v - n_pieces * _C2
    n_uniform = n_pieces // _NS
    n_extra = n_pieces - n_uniform * _NS
    stage_bufs = ((vals_a, sem_a, sem_c), (vals_b, sem_b, sem_d))

    def stage_in(k):
        vb, _, si = stage_bufs[k % 2]
        o = (k * _NS + sid) * _C2
        return pltpu.async_copy(w_hbm.at[pl.ds(o, _C2)], vb, si)

    incps = {0: stage_in(0), 1: stage_in(1)}

    o_x = (n_uniform * _NS + sid) * _C2
    o_r = n_pieces * _C2
    if n_extra > 0:
        @pl.when(sid < n_extra)
        def _():
            pltpu.async_copy(w_hbm.at[pl.ds(o_x, _C2)], vals_e, sem_e)
    if rem > 0:
        @pl.when(sid == _NS - 1)
        def _():
            pltpu.async_copy(w_hbm.at[pl.ds(o_r, rem)],
                             vals_e.at[pl.ds(0, rem)], sem_e)

    outcps = []
    for k in range(n_uniform):
        vb, so, _ = stage_bufs[k % 2]
        incps[k].wait()
        o = (k * _NS + sid) * _C2
        outcps.append(pltpu.async_copy(vb, w_sh.at[pl.ds(o, _C2)], so))
        if k + 2 < n_uniform:
            outcps[k].wait()
            incps[k + 2] = stage_in(k + 2)
    per_w = idx_hbm.shape[0] // _NW
    base = wid * per_w
    pre0 = pltpu.async_copy(idx_hbm.at[pl.ds(base, _C2)], idx_a, sem_c)

    for cp in outcps[-2:]:
        cp.wait()

    if n_extra > 0:
        @pl.when(sid < n_extra)
        def _():
            pltpu.make_async_copy(w_hbm.at[pl.ds(o_x, _C2)], vals_e,
                                  sem_e).wait()
            pltpu.async_copy(vals_e, w_sh.at[pl.ds(o_x, _C2)], sem_e).wait()

    if rem > 0:
        @pl.when(sid == _NS - 1)
        def _():
            pltpu.make_async_copy(w_hbm.at[pl.ds(o_r, rem)],
                                  vals_e.at[pl.ds(0, rem)], sem_e).wait()
            pltpu.async_copy(vals_e.at[pl.ds(0, rem)],
                             w_sh.at[pl.ds(o_r, rem)], sem_e).wait()
    plsc.subcore_barrier()

    nch = per_w // _C2
    bufs = ((idx_a, vals_a, sem_a), (idx_b, vals_b, sem_b))

    pre0.wait()
    pending = (pltpu.async_copy(w_sh.at[idx_a], vals_a, sem_a), base, vals_a)
    for g in range(1, nch):
        ib, vb, sm = bufs[g % 2]
        off = base + g * _C2
        pltpu.sync_copy(idx_hbm.at[pl.ds(off, _C2)], ib)
        cp = pltpu.async_copy(w_sh.at[ib], vb, sm)
        pcp, poff, pvb = pending
        pcp.wait()
        _sigmoid_chunk(pvb, _C2)
        pltpu.sync_copy(pvb, out_hbm.at[pl.ds(poff, _C2)])
        pending = (cp, off, vb)
    pcp, poff, pvb = pending
    pcp.wait()
    _sigmoid_chunk(pvb, _C2)
    pltpu.sync_copy(pvb, out_hbm.at[pl.ds(poff, _C2)])


def kernel(idx, weight):
    n = idx.shape[0]
    assert n % (_NW * _C2) == 0
    assert weight.shape[0] % 8 == 0
    flat_idx = idx.reshape(-1)

    run = pl.kernel(
        _sc_body,
        out_type=jax.ShapeDtypeStruct((n,), jnp.float32),
        mesh=plsc.VectorSubcoreMesh(core_axis_name="c", subcore_axis_name="s"),
        scratch_types=[
            pltpu.VMEM_SHARED((weight.shape[0],), jnp.float32),
            pltpu.VMEM((_C2,), jnp.int32),
            pltpu.VMEM((_C2,), jnp.int32),
            pltpu.VMEM((_C2,), jnp.float32),
            pltpu.VMEM((_C2,), jnp.float32),
            pltpu.VMEM((_C2,), jnp.float32),
            pltpu.SemaphoreType.DMA,
            pltpu.SemaphoreType.DMA,
            pltpu.SemaphoreType.DMA,
            pltpu.SemaphoreType.DMA,
            pltpu.SemaphoreType.DMA,
        ],
    )
    out = run(flat_idx, weight)
    return out.reshape(idx.shape)
